# Initial kernel scaffold; baseline (speedup 1.0000x reference)
import jax
import jax.numpy as jnp
from jax.experimental import pallas as pl


def kernel(x, edge_index, edge_attr, batch, W_node, b_node, W_edge, b_edge, convW1, convb1, convW2, convb2, Wf1, bf1, Wf2, bf2, Wf3, bf3):
    raise NotImplementedError("write your pallas kernel here")


# speedup vs baseline: 1.9392x; 1.9392x over previous
import functools

import jax
import jax.numpy as jnp
from jax import lax
from jax.experimental import pallas as pl
from jax.experimental.pallas import tpu as pltpu
from jax.experimental.pallas import tpu_sc as plsc

_N = 10000
_E = 320000
_HID = 128
_NC, _NS = 2, 16
_NW = _NC * _NS
_CHUNK = 128
_EPW = 10240
_EPAD = _NW * _EPW
_NCHUNKS = _EPW // _CHUNK
_NACC = 10240
_ROWS_PER_TILE = _NACC // _NS
_OUT_CHUNK = 128


def _mp_body(h_hbm, e_hbm, src_hbm, dst_hbm, agg_hbm,
             idx_src, idx_dst, ebuf, hbuf, acc, sem):
    cid = lax.axis_index("c")
    sid = lax.axis_index("s")
    wid = sid * _NC + cid

    def _zero_row(r, carry):
        for j in range(8):
            hbuf[r, pl.ds(j * 16, 16)] = jnp.zeros((16,), jnp.float32)
        return carry
    lax.fori_loop(0, _CHUNK, _zero_row, 0)
    rows_per_tile_acc = _NACC // _NS
    for q in range(rows_per_tile_acc // _CHUNK):
        pltpu.sync_copy(hbuf, acc.at[pl.ds(sid * rows_per_tile_acc + q * _CHUNK, _CHUNK)])
    plsc.subcore_barrier()

    base0 = wid * _EPW

    def _chunk(i, carry):
        base = base0 + i * _CHUNK
        ebase = jnp.minimum(base, _E - _CHUNK)
        pltpu.sync_copy(src_hbm.at[pl.ds(base, _CHUNK)], idx_src)
        pltpu.sync_copy(dst_hbm.at[pl.ds(base, _CHUNK)], idx_dst)
        pltpu.sync_copy(e_hbm.at[pl.ds(ebase, _CHUNK)], ebuf)
        pltpu.async_copy(h_hbm.at[idx_src], hbuf, sem).wait()

        def _row(r, c2):
            for j in range(8):
                sl = pl.ds(j * 16, 16)
                hbuf[r, sl] = jnp.maximum(hbuf[r, sl] + ebuf[r, sl], 0.0)
            return c2
        lax.fori_loop(0, _CHUNK, _row, 0)

        pltpu.sync_copy(hbuf, acc.at[idx_dst], add=True)
        return carry

    lax.fori_loop(0, _NCHUNKS, _chunk, 0)
    plsc.subcore_barrier()

    for q in range(_ROWS_PER_TILE // _OUT_CHUNK):
        r0 = sid * _ROWS_PER_TILE + q * _OUT_CHUNK
        pltpu.sync_copy(acc.at[pl.ds(r0, _OUT_CHUNK)], ebuf)
        pltpu.sync_copy(ebuf, agg_hbm.at[cid, pl.ds(r0, _OUT_CHUNK)])


_mp_kernel = pl.kernel(
    _mp_body,
    out_type=jax.ShapeDtypeStruct((_NC, _NACC, _HID), jnp.float32),
    mesh=plsc.VectorSubcoreMesh(core_axis_name="c", subcore_axis_name="s",
                                num_cores=_NC, num_subcores=_NS),
    scratch_types=[
        pltpu.VMEM((_CHUNK,), jnp.int32),
        pltpu.VMEM((_CHUNK,), jnp.int32),
        pltpu.VMEM((_CHUNK, _HID), jnp.float32),
        pltpu.VMEM((_CHUNK, _HID), jnp.float32),
        pltpu.VMEM_SHARED((_NACC, _HID), jnp.float32),
        pltpu.SemaphoreType.DMA,
    ],
)


def _linrelu_body(x_ref, w_ref, b_ref, o_ref):
    o_ref[:] = jnp.maximum(
        jnp.dot(x_ref[:], w_ref[:], preferred_element_type=jnp.float32)
        + b_ref[:], 0.0)


def _linrelu(x, w, b, blk):
    m, k = x.shape
    n = w.shape[1]
    return pl.pallas_call(
        _linrelu_body,
        grid=(m // blk,),
        in_specs=[
            pl.BlockSpec((blk, k), lambda i: (i, 0)),
            pl.BlockSpec((k, n), lambda i: (0, 0)),
            pl.BlockSpec((1, n), lambda i: (0, 0)),
        ],
        out_specs=pl.BlockSpec((blk, n), lambda i: (i, 0)),
        out_shape=jax.ShapeDtypeStruct((m, n), jnp.float32),
    )(x, w, b.reshape(1, n))


def _mlp_body(h_ref, a0_ref, a1_ref, w1_ref, b1_ref, w2_ref, b2_ref, o_ref,
              *, final_relu):
    z = h_ref[:] + a0_ref[0] + a1_ref[0]
    t = jnp.maximum(
        jnp.dot(z, w1_ref[:], preferred_element_type=jnp.float32)
        + b1_ref[:], 0.0)
    o = jnp.dot(t, w2_ref[:], preferred_element_type=jnp.float32) + b2_ref[:]
    if final_relu:
        o = jnp.maximum(o, 0.0)
    o_ref[:] = o


def _mlp(h, agg, w1, b1, w2, b2, final_relu):
    blk = 2000
    f = w1.shape[1]
    return pl.pallas_call(
        functools.partial(_mlp_body, final_relu=final_relu),
        grid=(_N // blk,),
        in_specs=[
            pl.BlockSpec((blk, _HID), lambda i: (i, 0)),
            pl.BlockSpec((1, blk, _HID), lambda i: (0, i, 0)),
            pl.BlockSpec((1, blk, _HID), lambda i: (1, i, 0)),
            pl.BlockSpec((_HID, f), lambda i: (0, 0)),
            pl.BlockSpec((1, f), lambda i: (0, 0)),
            pl.BlockSpec((f, _HID), lambda i: (0, 0)),
            pl.BlockSpec((1, _HID), lambda i: (0, 0)),
        ],
        out_specs=pl.BlockSpec((blk, _HID), lambda i: (i, 0)),
        out_shape=jax.ShapeDtypeStruct((_N, _HID), jnp.float32),
    )(h, agg, agg, w1, b1.reshape(1, f), w2, b2.reshape(1, _HID))


def _pool_ffn_body(h_ref, batch_ref, wf1_ref, bf1_ref, wf2_ref, bf2_ref,
                   wf3_ref, bf3_ref, o_ref, *, ng):
    gi = lax.broadcasted_iota(jnp.int32, (ng, _N), 0)
    onehot = (gi == batch_ref[:]).astype(jnp.float32)
    sums = jnp.dot(onehot, h_ref[:], preferred_element_type=jnp.float32)
    cnts = jnp.sum(onehot, axis=1, keepdims=True)
    pooled = sums / jnp.maximum(cnts, 1.0)
    o = jnp.maximum(
        jnp.dot(pooled, wf1_ref[:], preferred_element_type=jnp.float32)
        + bf1_ref[:], 0.0)
    o = jnp.maximum(
        jnp.dot(o, wf2_ref[:], preferred_element_type=jnp.float32)
        + bf2_ref[:], 0.0)
    o = jnp.dot(o, wf3_ref[:], preferred_element_type=jnp.float32) + bf3_ref[:]
    o_ref[:] = o


def _pool_ffn(h, batch, wf1, bf1, wf2, bf2, wf3, bf3):
    ng = 64
    ffn = wf1.shape[1]
    out = pl.pallas_call(
        functools.partial(_pool_ffn_body, ng=ng),
        out_shape=jax.ShapeDtypeStruct((ng, 1), jnp.float32),
    )(h, batch.reshape(1, _N), wf1, bf1.reshape(1, ffn),
      wf2, bf2.reshape(1, ffn), wf3, bf3.reshape(1, 1))
    return out.reshape(ng)


def kernel(x, edge_index, edge_attr, batch, W_node, b_node, W_edge, b_edge,
           convW1, convb1, convW2, convb2, Wf1, bf1, Wf2, bf2, Wf3, bf3):
    depth = convW1.shape[0]
    npad = _EPAD - _E
    src_p = jnp.concatenate([edge_index[0], jnp.zeros((npad,), jnp.int32)])
    dst_p = jnp.concatenate(
        [edge_index[1], _N + (jnp.arange(npad, dtype=jnp.int32) % (_NACC - _N))])

    h = _linrelu(x, W_node, b_node, blk=2000)
    e = _linrelu(edge_attr, W_edge, b_edge, blk=4000)

    for l in range(depth):
        agg = _mp_kernel(h, e, src_p, dst_p)
        h = _mlp(h, agg, convW1[l], convb1[l], convW2[l],
                 convb2[l], final_relu=(l < depth - 1))

    return _pool_ffn(h, batch, Wf1, bf1, Wf2, bf2, Wf3, bf3)
